# Initial kernel scaffold; baseline (speedup 1.0000x reference)
import jax
import jax.numpy as jnp
from jax.experimental import pallas as pl


def kernel(node_feats, edge_index, edge_feats, params):
    raise NotImplementedError("write your pallas kernel here")


# speedup vs baseline: 2.5613x; 2.5613x over previous
import functools

import jax
import jax.numpy as jnp
import numpy as np
from jax import lax
from jax.experimental import pallas as pl
from jax.experimental.pallas import tpu as pltpu
from jax.experimental.pallas import tpu_sc as plsc

D = 128
N = 10000
E = 320000

_EC2 = np.float32(np.float32(1.602176634e-19) ** 2)
_DEN1 = np.float32(np.float32(4.0 * np.pi) * np.float32(8.8541878128e-12))
_1EM10 = np.float32(1e-10)


def _mish(x):
    return x * jnp.tanh(jax.nn.softplus(x))


_BN = 2000


def _node_mlp_body(x_ref, ws1, bs1, ws2, bs2, wd1, bd1, wd2, bd2,
                   os_ref, od_ref):
    x = x_ref[...]
    hs = _mish(jnp.dot(x, ws1[...], preferred_element_type=jnp.float32)
               + bs1[...])
    os_ref[...] = (jnp.dot(hs, ws2[...], preferred_element_type=jnp.float32)
                   + bs2[...])
    hd = _mish(jnp.dot(x, wd1[...], preferred_element_type=jnp.float32)
               + bd1[...])
    od_ref[...] = (jnp.dot(hd, wd2[...], preferred_element_type=jnp.float32)
                   + bd2[...])


def _node_mlps(node_feats, p):
    w_spec = pl.BlockSpec((D, D), lambda i: (0, 0))
    b_spec = pl.BlockSpec((1, D), lambda i: (0, 0))
    x_spec = pl.BlockSpec((_BN, D), lambda i: (i, 0))
    return pl.pallas_call(
        _node_mlp_body,
        grid=(N // _BN,),
        in_specs=[x_spec,
                  w_spec, b_spec, w_spec, b_spec,
                  w_spec, b_spec, w_spec, b_spec],
        out_specs=[x_spec, x_spec],
        out_shape=[jax.ShapeDtypeStruct((N, D), jnp.float32)] * 2,
    )(node_feats,
      p["src_W1"], p["src_b1"].reshape(1, D), p["src_W2"], p["src_b2"].reshape(1, D),
      p["dst_W1"], p["dst_b1"].reshape(1, D), p["dst_W2"], p["dst_b2"].reshape(1, D))


_NC = 2
_NS = 16
_NW = _NC * _NS
_EPW = E // _NW
_GCH = 80
_GCHUNKS = _EPW // _GCH


def _sc_gather_body(hs_hbm, hd_hbm, src_hbm, dst_hbm, os_hbm, od_hbm,
                    si_v, di_v, rs_v, rd_v, sem_s, sem_d):
    wid = lax.axis_index("s") * _NC + lax.axis_index("c")
    base = wid * _EPW

    def chunk(i, carry):
        off = pl.multiple_of(base + i * _GCH, _GCH)
        pltpu.sync_copy(src_hbm.at[pl.ds(off, _GCH)], si_v)
        pltpu.sync_copy(dst_hbm.at[pl.ds(off, _GCH)], di_v)
        cs = pltpu.async_copy(hs_hbm.at[si_v], rs_v, sem_s)
        cd = pltpu.async_copy(hd_hbm.at[di_v], rd_v, sem_d)
        cs.wait()
        cd.wait()
        pltpu.sync_copy(rs_v, os_hbm.at[pl.ds(off, _GCH)])
        pltpu.sync_copy(rd_v, od_hbm.at[pl.ds(off, _GCH)])
        return carry

    lax.fori_loop(0, _GCHUNKS, chunk, 0)


def _sc_gather(h_src, h_dst, src, dst):
    mesh = plsc.VectorSubcoreMesh(core_axis_name="c", subcore_axis_name="s")
    f = pl.kernel(
        _sc_gather_body,
        mesh=mesh,
        out_type=[jax.ShapeDtypeStruct((E, D), jnp.float32)] * 2,
        scratch_types=[
            pltpu.VMEM((_GCH,), jnp.int32),
            pltpu.VMEM((_GCH,), jnp.int32),
            pltpu.VMEM((_GCH, D), jnp.float32),
            pltpu.VMEM((_GCH, D), jnp.float32),
            pltpu.SemaphoreType.DMA,
            pltpu.SemaphoreType.DMA,
        ],
    )
    return f(h_src, h_dst, src, dst)


_BE = 2000


def _edge_msg_body(ef_ref, hs_ref, hd_ref,
                   we1, be1, we2, be2, wm1, bm1, wm2, bm2,
                   m_ref):
    ef_in = ef_ref[...]
    h1 = _mish(jnp.dot(ef_in, we1[...], preferred_element_type=jnp.float32)
               + be1[...])
    ef = jnp.dot(h1, we2[...], preferred_element_type=jnp.float32) + be2[...]
    t = hs_ref[...] * hd_ref[...]
    hn = t * _EC2 / (_DEN1 * ef * _1EM10)
    h2 = _mish(jnp.dot(hn, wm1[...], preferred_element_type=jnp.float32)
               + bm1[...])
    m_ref[...] = (jnp.dot(h2, wm2[...], preferred_element_type=jnp.float32)
                  + bm2[...])


def _edge_messages(edge_feats, hs_g, hd_g, p):
    w_spec = pl.BlockSpec((D, D), lambda i: (0, 0))
    b_spec = pl.BlockSpec((1, D), lambda i: (0, 0))
    e_spec = pl.BlockSpec((_BE, D), lambda i: (i, 0))
    return pl.pallas_call(
        _edge_msg_body,
        grid=(E // _BE,),
        in_specs=[e_spec, e_spec, e_spec,
                  w_spec, b_spec, w_spec, b_spec,
                  w_spec, b_spec, w_spec, b_spec],
        out_specs=e_spec,
        out_shape=jax.ShapeDtypeStruct((E, D), jnp.float32),
    )(edge_feats, hs_g, hd_g,
      p["edge_W1"], p["edge_b1"].reshape(1, D), p["edge_W2"], p["edge_b2"].reshape(1, D),
      p["m_W1"], p["m_b1"].reshape(1, D), p["m_W2"], p["m_b2"].reshape(1, D))


_SCH = 80
_SCHUNKS = _EPW // _SCH
_NPAD = 10240
_RPT = _NPAD // _NS
_RCH = 128
_RCHUNKS = _RPT // _RCH


def _sc_scatter_body(m_hbm, dst_hbm, out_hbm, acc_sh, mv, iv, rowbuf):
    cid = lax.axis_index("c")
    sid = lax.axis_index("s")

    def zrow(i, carry):
        for j in range(D // 16):
            rowbuf[i, pl.ds(j * 16, 16)] = jnp.zeros((16,), jnp.float32)
        return carry

    lax.fori_loop(0, _RCH, zrow, 0)
    for j in range(_RCHUNKS):
        pltpu.sync_copy(rowbuf, acc_sh.at[pl.ds(sid * _RPT + j * _RCH, _RCH)])
    plsc.subcore_barrier()

    base = (cid * _NS + sid) * _EPW

    def chunk(i, carry):
        off = pl.multiple_of(base + i * _SCH, _SCH)
        pltpu.sync_copy(m_hbm.at[pl.ds(off, _SCH)], mv)
        pltpu.sync_copy(dst_hbm.at[pl.ds(off, _SCH)], iv)
        pltpu.sync_copy(mv, acc_sh.at[iv], add=True)
        return carry

    lax.fori_loop(0, _SCHUNKS, chunk, 0)
    plsc.subcore_barrier()

    for j in range(_RCHUNKS):
        r0 = sid * _RPT + j * _RCH
        pltpu.sync_copy(acc_sh.at[pl.ds(r0, _RCH)], rowbuf)
        pltpu.sync_copy(rowbuf, out_hbm.at[pl.ds(cid * _NPAD + r0, _RCH)])


def _sc_scatter(m, dst):
    mesh = plsc.VectorSubcoreMesh(core_axis_name="c", subcore_axis_name="s")
    f = pl.kernel(
        _sc_scatter_body,
        mesh=mesh,
        out_type=jax.ShapeDtypeStruct((2 * _NPAD, D), jnp.float32),
        scratch_types=[
            pltpu.VMEM_SHARED((_NPAD, D), jnp.float32),
            pltpu.VMEM((_SCH, D), jnp.float32),
            pltpu.VMEM((_SCH,), jnp.int32),
            pltpu.VMEM((_RCH, D), jnp.float32),
        ],
    )
    return f(m, dst)


def _final_body(x_ref, h0_ref, h1_ref, o_ref):
    o_ref[...] = jax.nn.softplus(x_ref[...] + (h0_ref[...] + h1_ref[...]))


def _final(node_feats, h2):
    x_spec = pl.BlockSpec((_BN, D), lambda i: (i, 0))
    return pl.pallas_call(
        _final_body,
        grid=(N // _BN,),
        in_specs=[x_spec, x_spec, x_spec],
        out_specs=x_spec,
        out_shape=jax.ShapeDtypeStruct((N, D), jnp.float32),
    )(node_feats, h2[:N], h2[_NPAD:_NPAD + N])


def kernel(node_feats, edge_index, edge_feats, params):
    src = edge_index[0].astype(jnp.int32)
    dst = edge_index[1].astype(jnp.int32)
    h_src, h_dst = _node_mlps(node_feats, params)
    hs_g, hd_g = _sc_gather(h_src, h_dst, src, dst)
    m = _edge_messages(edge_feats, hs_g, hd_g, params)
    h2 = _sc_scatter(m, dst)
    return _final(node_feats, h2)
